# baseline (speedup 1.0000x reference)
import functools

import jax
import jax.numpy as jnp
from jax import lax
from jax.experimental import pallas as pl
from jax.experimental.pallas import tpu as pltpu
from jax.experimental.pallas import tpu_sc as plsc

VOCAB_SIZE = 1000000
EMBED_DIM = 16
BATCH = 16384
MAX_NORM = 2.0

NUM_CORES = 2
NUM_SUBCORES = 16
NUM_WORKERS = NUM_CORES * NUM_SUBCORES
ROWS_PER_WORKER = BATCH // NUM_WORKERS
CHUNK = 128
CHUNKS_PER_WORKER = ROWS_PER_WORKER // CHUNK
BLOCKS_PER_WORKER = ROWS_PER_WORKER // 16


def _rsqrt(x):
    i = lax.bitcast_convert_type(x, jnp.int32)
    y = lax.bitcast_convert_type(jnp.int32(0x5F3759DF) - (i >> 1), jnp.float32)
    for _ in range(3):
        y = y * (1.5 - 0.5 * x * y * y)
    return y


def _sc_embed(table, idx2d):
    mesh = plsc.VectorSubcoreMesh(core_axis_name="c", subcore_axis_name="s")

    @functools.partial(
        pl.kernel,
        out_type=jax.ShapeDtypeStruct((BATCH, EMBED_DIM), jnp.float32),
        mesh=mesh,
        compiler_params=pltpu.CompilerParams(
            needs_layout_passes=False, use_tc_tiling_on_sc=False
        ),
        scratch_types=[
            pltpu.VMEM((CHUNKS_PER_WORKER, CHUNK), jnp.int32),
            pltpu.VMEM((ROWS_PER_WORKER, EMBED_DIM), jnp.float32),
            pltpu.SemaphoreType.DMA,
        ],
    )
    def k(table_hbm, idx_hbm, out_hbm, idx_v, rows_v, sem):
        wid = lax.axis_index("s") * NUM_CORES + lax.axis_index("c")
        pltpu.sync_copy(
            idx_hbm.at[pl.ds(wid * CHUNKS_PER_WORKER, CHUNKS_PER_WORKER)], idx_v
        )
        copies = [
            pltpu.async_copy(
                table_hbm.at[idx_v.at[j]],
                rows_v.at[pl.ds(j * CHUNK, CHUNK)],
                sem,
            )
            for j in range(CHUNKS_PER_WORKER)
        ]
        for c in copies:
            c.wait()

        lanes = lax.iota(jnp.int32, 16)

        def block(b, carry):
            rid = b * 16 + lanes
            cols = []
            ss = jnp.zeros((16,), jnp.float32)
            for c in range(EMBED_DIM):
                col = plsc.load_gather(rows_v, [rid, jnp.full((16,), c, jnp.int32)])
                cols.append(col)
                ss = ss + col * col
            scale = jnp.where(ss > MAX_NORM * MAX_NORM, MAX_NORM * _rsqrt(ss), 1.0)
            for c in range(EMBED_DIM):
                plsc.store_scatter(
                    rows_v, [rid, jnp.full((16,), c, jnp.int32)], cols[c] * scale
                )
            return carry

        lax.fori_loop(0, BLOCKS_PER_WORKER, block, None)

        pltpu.sync_copy(
            rows_v, out_hbm.at[pl.ds(wid * ROWS_PER_WORKER, ROWS_PER_WORKER)]
        )

    return k(table, idx2d)


def kernel(data, table):
    idx2d = data.reshape(NUM_WORKERS * CHUNKS_PER_WORKER, CHUNK)
    return _sc_embed(table, idx2d)
